# baseline (speedup 1.0000x reference)
import jax
import jax.numpy as jnp
from jax.experimental import pallas as pl
from jax.experimental.pallas import tpu as pltpu

_N = 8192
_K = 4096
_ROWS = 128
_BLOCK_ROWS = 32


def _count_ge(key, cand):
    accs = [None] * 8
    for t in range(key.shape[1] // (8 * 128)):
        for j in range(8):
            s = (t * 8 + j) * 128
            part = (key[:, s:s + 128] >= cand).astype(jnp.int32)
            accs[j] = part if accs[j] is None else accs[j] + part
    while len(accs) > 1:
        accs = [a + b for a, b in zip(accs[::2], accs[1::2])]
    return jnp.sum(accs[0], axis=1, keepdims=True)


def _sdd_block(x_ref, o_ref):
    int_max = jnp.int32(2**31 - 1)
    int_min = jnp.int32(-(2**31))
    x = x_ref[...]
    xz = x + 0.0
    b = jax.lax.bitcast_convert_type(xz, jnp.int32)
    key = jnp.where(b >= 0, b, int_max - b)

    cnt_pos = _count_ge(key, jnp.zeros_like(key[:, :1]))
    pos = cnt_pos >= _K
    prefix = jnp.where(pos, jnp.int32(0), int_min)
    cnt0 = jnp.where(pos, cnt_pos, jnp.int32(_N))

    def sweep(i, prefix, cntp):
        bit = jnp.left_shift(jnp.int32(1), jnp.int32(30) - i)
        cand = prefix + bit
        cnt = _count_ge(key, cand)
        take = cnt >= _K
        return jnp.where(take, cand, prefix), jnp.where(take, cnt, cntp)

    def cond(state):
        i, _, cntp = state
        return jnp.logical_and(i < 28, jnp.any(cntp > _K))

    def body(state):
        i, prefix, cntp = state
        for j in range(4):
            prefix, cntp = sweep(i + jnp.int32(j), prefix, cntp)
        return (i + jnp.int32(4), prefix, cntp)

    i, prefix, cntp = jax.lax.while_loop(
        cond, body, (jnp.int32(0), prefix, cnt0))
    for j in range(3):
        prefix, cntp = sweep(jnp.int32(28 + j), prefix, cntp)

    mask = key >= prefix
    o_ref[...] = jnp.where(mask, x, jnp.float32(0.0))


def kernel(x):
    return pl.pallas_call(
        _sdd_block,
        out_shape=jax.ShapeDtypeStruct(x.shape, x.dtype),
        grid=(_ROWS // _BLOCK_ROWS,),
        in_specs=[pl.BlockSpec((_BLOCK_ROWS, _N), lambda i: (i, 0))],
        out_specs=pl.BlockSpec((_BLOCK_ROWS, _N), lambda i: (i, 0)),
        compiler_params=pltpu.CompilerParams(
            dimension_semantics=("parallel",)
        ),
    )(x)


# speedup vs baseline: 1.3961x; 1.3961x over previous
import jax
import jax.numpy as jnp
from jax.experimental import pallas as pl
from jax.experimental.pallas import tpu as pltpu

_N = 8192
_K = 4096
_ROWS = 128
_GROUP_ROWS = 32
_GROUPS_PER_BLOCK = 2
_BLOCK_ROWS = _GROUP_ROWS * _GROUPS_PER_BLOCK


def _count_ge(key, cand):
    one = jnp.float32(1.0)
    zero = jnp.float32(0.0)
    accs = [None] * 4
    for t in range(key.shape[1] // (4 * 128)):
        for j in range(4):
            s = (t * 4 + j) * 128
            part = jnp.where(key[:, s:s + 128] >= cand, one, zero)
            accs[j] = part if accs[j] is None else accs[j] + part
    while len(accs) > 1:
        accs = [a + b for a, b in zip(accs[::2], accs[1::2])]
    return jnp.sum(accs[0], axis=1, keepdims=True)


def _sdd_block(x_ref, o_ref):
    int_max = jnp.int32(2**31 - 1)
    int_min = jnp.int32(-(2**31))
    kf = jnp.float32(_K)
    g_rows = _GROUP_ROWS
    n_groups = _GROUPS_PER_BLOCK

    keys = []
    xs = []
    for g in range(n_groups):
        x = x_ref[g * g_rows:(g + 1) * g_rows, :]
        xz = x + 0.0
        b = jax.lax.bitcast_convert_type(xz, jnp.int32)
        keys.append(jnp.where(b >= 0, b, int_max - b))
        xs.append(x)

    prefixes = []
    cntps = []
    for g in range(n_groups):
        cnt_pos = _count_ge(keys[g], jnp.zeros_like(keys[g][:, :1]))
        pos = cnt_pos >= kf
        prefixes.append(jnp.where(pos, jnp.int32(0), int_min))
        cntps.append(jnp.where(pos, cnt_pos, jnp.float32(_N)))

    def sweep(i, prefix, cntp, key):
        bit = jnp.left_shift(jnp.int32(1), jnp.int32(30) - i)
        cand = prefix + bit
        cnt = _count_ge(key, cand)
        take = cnt >= kf
        return jnp.where(take, cand, prefix), jnp.where(take, cnt, cntp)

    def cond(state):
        done = jnp.bool_(False)
        for g in range(n_groups):
            done = jnp.logical_or(done, jnp.any(state[1 + 2 * g + 1] > kf))
        return jnp.logical_and(state[0] < 28, done)

    def body(state):
        i = state[0]
        ps = list(state[1::2][:n_groups])
        cs = list(state[2::2][:n_groups])
        for j in range(4):
            for g in range(n_groups):
                ps[g], cs[g] = sweep(i + jnp.int32(j), ps[g], cs[g], keys[g])
        out = [i + jnp.int32(4)]
        for g in range(n_groups):
            out.extend((ps[g], cs[g]))
        return tuple(out)

    state = [jnp.int32(0)]
    for g in range(n_groups):
        state.extend((prefixes[g], cntps[g]))
    state = jax.lax.while_loop(cond, body, tuple(state))
    ps = list(state[1::2][:n_groups])
    cs = list(state[2::2][:n_groups])
    for j in range(3):
        for g in range(n_groups):
            ps[g], cs[g] = sweep(jnp.int32(28 + j), ps[g], cs[g], keys[g])

    for g in range(n_groups):
        mask = keys[g] >= ps[g]
        o_ref[g * g_rows:(g + 1) * g_rows, :] = jnp.where(
            mask, xs[g], jnp.float32(0.0))


def kernel(x):
    return pl.pallas_call(
        _sdd_block,
        out_shape=jax.ShapeDtypeStruct(x.shape, x.dtype),
        grid=(_ROWS // _BLOCK_ROWS,),
        in_specs=[pl.BlockSpec((_BLOCK_ROWS, _N), lambda i: (i, 0))],
        out_specs=pl.BlockSpec((_BLOCK_ROWS, _N), lambda i: (i, 0)),
        compiler_params=pltpu.CompilerParams(
            dimension_semantics=("parallel",)
        ),
    )(x)
